# baseline (speedup 1.0000x reference)
import functools

import jax
import jax.numpy as jnp
from jax import lax
from jax.experimental import pallas as pl
from jax.experimental.pallas import tpu as pltpu
from jax.experimental.pallas import tpu_sc as plsc

_LATENT_DIM = 64
_NUM_CODES = 1024
_BLOCK_ROWS = 4096


def _argmin_body(x_ref, emb_ref, idx_ref, embt_ref):
    xb = x_ref[...]
    emb = emb_ref[...]
    sim = jnp.dot(xb, emb, preferred_element_type=jnp.float32)
    e2 = jnp.sum(emb * emb, axis=0, keepdims=True)
    scores = e2 - 2.0 * sim
    idx_ref[0, 0, :] = jnp.argmin(scores, axis=1)

    @pl.when(pl.program_id(0) == 0)
    def _():
        embt_ref[...] = emb.T


def _codebook_gather(table, idx):
    info = plsc.get_sparse_core_info()
    nw = info.num_cores * info.num_subcores
    b = idx.shape[0]
    d = table.shape[1]
    b_per_w = b // nw
    mesh = plsc.VectorSubcoreMesh(core_axis_name="c", subcore_axis_name="s")

    @functools.partial(
        pl.kernel,
        mesh=mesh,
        out_type=jax.ShapeDtypeStruct((b, d), table.dtype),
        compiler_params=pltpu.CompilerParams(use_tc_tiling_on_sc=False),
        scratch_types=[
            pltpu.VMEM((b_per_w,), jnp.int32),
            pltpu.VMEM((b_per_w, d), jnp.float32),
            pltpu.SemaphoreType.DMA,
        ],
    )
    def gath(table_hbm, idx_hbm, out_hbm, idx_v, rows_v, sem):
        wid = lax.axis_index("s") * info.num_cores + lax.axis_index("c")
        base = wid * b_per_w
        pltpu.sync_copy(idx_hbm.at[pl.ds(base, b_per_w)], idx_v)
        pltpu.async_copy(table_hbm.at[idx_v], rows_v, sem).wait()
        pltpu.sync_copy(rows_v, out_hbm.at[pl.ds(base, b_per_w)])

    return gath(table, idx)


@jax.jit
def kernel(x, embeddings):
    orig_shape = x.shape
    xf = x.reshape(-1, _LATENT_DIM)
    rows = xf.shape[0]
    nb = rows // _BLOCK_ROWS
    idx3, embt = pl.pallas_call(
        _argmin_body,
        grid=(nb,),
        in_specs=[
            pl.BlockSpec((_BLOCK_ROWS, _LATENT_DIM), lambda i: (i, 0)),
            pl.BlockSpec((_LATENT_DIM, _NUM_CODES), lambda i: (0, 0)),
        ],
        out_specs=[
            pl.BlockSpec((1, 1, _BLOCK_ROWS), lambda i: (i, 0, 0)),
            pl.BlockSpec((_NUM_CODES, _LATENT_DIM), lambda i: (0, 0)),
        ],
        out_shape=[
            jax.ShapeDtypeStruct((nb, 1, _BLOCK_ROWS), jnp.int32),
            jax.ShapeDtypeStruct((_NUM_CODES, _LATENT_DIM), jnp.float32),
        ],
    )(xf, embeddings)
    quant = _codebook_gather(embt, idx3.reshape(-1))
    return quant.reshape(orig_shape)


# speedup vs baseline: 2.1076x; 2.1076x over previous
import functools

import jax
import jax.numpy as jnp
from jax.experimental import pallas as pl

_LATENT_DIM = 64
_NUM_CODES = 1024
_BLOCK_ROWS = 4096
_SUB_ROWS = 1024


def _vq_body(x_ref, emb_ref, o_ref):
    emb = emb_ref[...]
    e2 = jnp.sum(emb * emb, axis=0, keepdims=True)
    for k in range(_BLOCK_ROWS // _SUB_ROWS):
        xb = x_ref[pl.ds(k * _SUB_ROWS, _SUB_ROWS), :]
        sim = jnp.dot(xb, emb, preferred_element_type=jnp.float32)
        scores = e2 - 2.0 * sim
        idx = jnp.argmin(scores, axis=1).reshape(-1, 1)
        col = jax.lax.broadcasted_iota(jnp.int32, scores.shape, 1)
        onehot = (col == idx).astype(jnp.float32)
        o_ref[pl.ds(k * _SUB_ROWS, _SUB_ROWS), :] = jax.lax.dot_general(
            onehot, emb, (((1,), (1,)), ((), ())),
            preferred_element_type=jnp.float32)


@functools.partial(jax.jit, static_argnames=("interpret",))
def kernel(x, embeddings, interpret=False):
    orig_shape = x.shape
    xf = x.reshape(-1, _LATENT_DIM)
    rows = xf.shape[0]
    grid = (rows // _BLOCK_ROWS,)
    out = pl.pallas_call(
        _vq_body,
        grid=grid,
        in_specs=[
            pl.BlockSpec((_BLOCK_ROWS, _LATENT_DIM), lambda i: (i, 0)),
            pl.BlockSpec((_LATENT_DIM, _NUM_CODES), lambda i: (0, 0)),
        ],
        out_specs=pl.BlockSpec((_BLOCK_ROWS, _LATENT_DIM), lambda i: (i, 0)),
        out_shape=jax.ShapeDtypeStruct((rows, _LATENT_DIM), jnp.float32),
        interpret=interpret,
    )(xf, embeddings)
    return out.reshape(orig_shape)


# speedup vs baseline: 2.1823x; 1.0354x over previous
import functools

import jax
import jax.numpy as jnp
from jax.experimental import pallas as pl

_LATENT_DIM = 64
_NUM_CODES = 1024
_BLOCK_ROWS = 8192
_SUB_ROWS = 1024


def _vq_body(x_ref, emb_ref, o_ref):
    emb = emb_ref[...]
    e2 = jnp.sum(emb * emb, axis=0, keepdims=True)
    for k in range(_BLOCK_ROWS // _SUB_ROWS):
        xb = x_ref[pl.ds(k * _SUB_ROWS, _SUB_ROWS), :]
        sim = jnp.dot(xb, emb, preferred_element_type=jnp.float32)
        scores = e2 - 2.0 * sim
        idx = jnp.argmin(scores, axis=1).reshape(-1, 1)
        col = jax.lax.broadcasted_iota(jnp.int32, scores.shape, 1)
        onehot = (col == idx).astype(jnp.float32)
        o_ref[pl.ds(k * _SUB_ROWS, _SUB_ROWS), :] = jax.lax.dot_general(
            onehot, emb, (((1,), (1,)), ((), ())),
            preferred_element_type=jnp.float32)


@functools.partial(jax.jit, static_argnames=("interpret",))
def kernel(x, embeddings, interpret=False):
    orig_shape = x.shape
    xf = x.reshape(-1, _LATENT_DIM)
    rows = xf.shape[0]
    grid = (rows // _BLOCK_ROWS,)
    out = pl.pallas_call(
        _vq_body,
        grid=grid,
        in_specs=[
            pl.BlockSpec((_BLOCK_ROWS, _LATENT_DIM), lambda i: (i, 0)),
            pl.BlockSpec((_LATENT_DIM, _NUM_CODES), lambda i: (0, 0)),
        ],
        out_specs=pl.BlockSpec((_BLOCK_ROWS, _LATENT_DIM), lambda i: (i, 0)),
        out_shape=jax.ShapeDtypeStruct((rows, _LATENT_DIM), jnp.float32),
        interpret=interpret,
    )(xf, embeddings)
    return out.reshape(orig_shape)


# speedup vs baseline: 2.2291x; 1.0214x over previous
import functools

import jax
import jax.numpy as jnp
from jax.experimental import pallas as pl

_LATENT_DIM = 64
_NUM_CODES = 1024
_BLOCK_ROWS = 8192
_SUB_ROWS = 512


def _vq_body(x_ref, emb_ref, o_ref):
    emb = emb_ref[...]
    e2 = jnp.sum(emb * emb, axis=0, keepdims=True)
    for k in range(_BLOCK_ROWS // _SUB_ROWS):
        xb = x_ref[pl.ds(k * _SUB_ROWS, _SUB_ROWS), :]
        sim = jnp.dot(xb, emb, preferred_element_type=jnp.float32)
        scores = e2 - 2.0 * sim
        idx = jnp.argmin(scores, axis=1).reshape(-1, 1)
        col = jax.lax.broadcasted_iota(jnp.int32, scores.shape, 1)
        onehot = (col == idx).astype(jnp.float32)
        o_ref[pl.ds(k * _SUB_ROWS, _SUB_ROWS), :] = jax.lax.dot_general(
            onehot, emb, (((1,), (1,)), ((), ())),
            preferred_element_type=jnp.float32)


@functools.partial(jax.jit, static_argnames=("interpret",))
def kernel(x, embeddings, interpret=False):
    orig_shape = x.shape
    xf = x.reshape(-1, _LATENT_DIM)
    rows = xf.shape[0]
    grid = (rows // _BLOCK_ROWS,)
    out = pl.pallas_call(
        _vq_body,
        grid=grid,
        in_specs=[
            pl.BlockSpec((_BLOCK_ROWS, _LATENT_DIM), lambda i: (i, 0)),
            pl.BlockSpec((_LATENT_DIM, _NUM_CODES), lambda i: (0, 0)),
        ],
        out_specs=pl.BlockSpec((_BLOCK_ROWS, _LATENT_DIM), lambda i: (i, 0)),
        out_shape=jax.ShapeDtypeStruct((rows, _LATENT_DIM), jnp.float32),
        interpret=interpret,
    )(xf, embeddings)
    return out.reshape(orig_shape)


# speedup vs baseline: 2.3290x; 1.0448x over previous
import functools

import jax
import jax.numpy as jnp
from jax.experimental import pallas as pl

_LATENT_DIM = 64
_NUM_CODES = 1024
_BLOCK_ROWS = 8192
_SUB_ROWS = 256


def _vq_body(x_ref, emb_ref, o_ref):
    emb = emb_ref[...]
    e2 = jnp.sum(emb * emb, axis=0, keepdims=True)
    for k in range(_BLOCK_ROWS // _SUB_ROWS):
        xb = x_ref[pl.ds(k * _SUB_ROWS, _SUB_ROWS), :]
        sim = jnp.dot(xb, emb, preferred_element_type=jnp.float32)
        scores = e2 - 2.0 * sim
        idx = jnp.argmin(scores, axis=1).reshape(-1, 1)
        col = jax.lax.broadcasted_iota(jnp.int32, scores.shape, 1)
        onehot = (col == idx).astype(jnp.float32)
        o_ref[pl.ds(k * _SUB_ROWS, _SUB_ROWS), :] = jax.lax.dot_general(
            onehot, emb, (((1,), (1,)), ((), ())),
            preferred_element_type=jnp.float32)


@functools.partial(jax.jit, static_argnames=("interpret",))
def kernel(x, embeddings, interpret=False):
    orig_shape = x.shape
    xf = x.reshape(-1, _LATENT_DIM)
    rows = xf.shape[0]
    grid = (rows // _BLOCK_ROWS,)
    out = pl.pallas_call(
        _vq_body,
        grid=grid,
        in_specs=[
            pl.BlockSpec((_BLOCK_ROWS, _LATENT_DIM), lambda i: (i, 0)),
            pl.BlockSpec((_LATENT_DIM, _NUM_CODES), lambda i: (0, 0)),
        ],
        out_specs=pl.BlockSpec((_BLOCK_ROWS, _LATENT_DIM), lambda i: (i, 0)),
        out_shape=jax.ShapeDtypeStruct((rows, _LATENT_DIM), jnp.float32),
        interpret=interpret,
    )(xf, embeddings)
    return out.reshape(orig_shape)
